# baseline (speedup 1.0000x reference)
import functools

import jax
import jax.numpy as jnp
from jax import lax
from jax.experimental import pallas as pl
from jax.experimental.pallas import tpu as pltpu
from jax.experimental.pallas import tpu_sc as plsc

_D = 128
_QUEUE = 65536
_INV_T = 10.0
_B = 256
_W = 8192
_HW = 2048
_NBLK = _QUEUE // _HW
_NCOL = 1 + 2048 + _QUEUE
_GRID = (_NCOL + _W - 1) // _W


def _out_body(q_ref, k_ref, pos_ref, neg_ref, memA_ref, memB_ref, memC_ref,
              memD_ref, out_ref, stash_ref):
    t = pl.program_id(0)
    q = q_ref[:]
    is0 = t == 0

    @pl.when(is0)
    def _():
        l_pos_k = jnp.sum(q * k_ref[:], axis=1, keepdims=True) * _INV_T
        l_pos_set = jnp.mean(
            jnp.sum(q[:, None, :] * pos_ref[:], axis=2), axis=1, keepdims=True
        ) * _INV_T
        stash_ref[:, 1:2] = l_pos_k
        stash_ref[:, 2:3] = l_pos_set

    srcA = jnp.where(is0, neg_ref[:], memA_ref[:])
    dn = (((1,), (1,)), ((), ()))
    pA = lax.dot_general(
        q, srcA, dn, preferred_element_type=jnp.float32) * _INV_T
    pB = lax.dot_general(
        q, memB_ref[:], dn, preferred_element_type=jnp.float32) * _INV_T
    pC = lax.dot_general(
        q, memC_ref[:], dn, preferred_element_type=jnp.float32) * _INV_T
    pD = lax.dot_general(
        q, memD_ref[:], dn, preferred_element_type=jnp.float32) * _INV_T
    p = jnp.concatenate([pA, pB, pC, pD], axis=1)
    rolled = pltpu.roll(p, shift=1, axis=1)

    out_ref[0:_B, :] = rolled
    out_ref[_B:2 * _B, :] = rolled
    prev = stash_ref[:, 0:1]
    out_ref[0:_B, 0:1] = jnp.where(is0, stash_ref[:, 1:2], prev)
    out_ref[_B:2 * _B, 0:1] = jnp.where(is0, stash_ref[:, 2:3], prev)
    stash_ref[:, 0:1] = rolled[:, 0:1]


def _logits(q, k, pos_set, neg_flat, memory):
    return pl.pallas_call(
        _out_body,
        grid=(_GRID,),
        in_specs=[
            pl.BlockSpec((_B, _D), lambda t: (0, 0)),
            pl.BlockSpec((_B, _D), lambda t: (0, 0)),
            pl.BlockSpec((_B, 4, _D), lambda t: (0, 0, 0)),
            pl.BlockSpec((_HW, _D), lambda t: (0, 0)),
            pl.BlockSpec(
                (_HW, _D), lambda t: (jnp.maximum(4 * t - 1, 3), 0)
            ),
            pl.BlockSpec(
                (_HW, _D), lambda t: (jnp.minimum(4 * t, _NBLK - 4), 0)
            ),
            pl.BlockSpec(
                (_HW, _D), lambda t: (jnp.minimum(4 * t + 1, _NBLK - 3), 0)
            ),
            pl.BlockSpec(
                (_HW, _D), lambda t: (jnp.minimum(4 * t + 2, _NBLK - 2), 0)
            ),
        ],
        out_specs=pl.BlockSpec((2 * _B, _W), lambda t: (0, t)),
        out_shape=jax.ShapeDtypeStruct((2 * _B, _NCOL), jnp.float32),
        scratch_shapes=[pltpu.VMEM((_B, _D), jnp.float32)],
    )(q, k, pos_set, neg_flat, memory, memory, memory, memory)


def _queue_update(memory, k_all):
    info = plsc.get_sparse_core_info()
    nw = info.num_cores * info.num_subcores
    rows_per = _QUEUE // nw
    ch = 256
    nch = rows_per // ch
    mesh = plsc.VectorSubcoreMesh(core_axis_name="c", subcore_axis_name="s")

    @functools.partial(
        pl.kernel,
        mesh=mesh,
        out_type=jax.ShapeDtypeStruct((_QUEUE, _D), jnp.float32),
        scratch_types=[
            pltpu.VMEM((ch, _D), jnp.float32),
            pltpu.VMEM((ch, _D), jnp.float32),
            pltpu.SemaphoreType.DMA,
            pltpu.SemaphoreType.DMA,
            pltpu.SemaphoreType.DMA,
            pltpu.SemaphoreType.DMA,
        ],
    )
    def body(mem_hbm, kall_hbm, out_hbm, buf0, buf1, si0, si1, so0, so1):
        wid = lax.axis_index("s") * info.num_cores + lax.axis_index("c")
        base = wid * rows_per
        bufs = (buf0, buf1)
        sin = (si0, si1)
        sout = (so0, so1)

        def start_in(c):
            b = bufs[c % 2]
            sem = sin[c % 2]
            if c * ch < _B:
                @pl.when(wid == 0)
                def _():
                    pltpu.make_async_copy(
                        kall_hbm.at[pl.ds(c * ch, ch)], b, sem
                    ).start()

                @pl.when(wid != 0)
                def _():
                    pltpu.make_async_copy(
                        mem_hbm.at[pl.ds(base + c * ch, ch)], b, sem
                    ).start()
            else:
                pltpu.make_async_copy(
                    mem_hbm.at[pl.ds(base + c * ch, ch)], b, sem
                ).start()

        start_in(0)
        for c in range(nch):
            b = bufs[c % 2]
            pltpu.make_async_copy(
                mem_hbm.at[pl.ds(base + c * ch, ch)], b, sin[c % 2]
            ).wait()
            out_cp = pltpu.make_async_copy(
                b, out_hbm.at[pl.ds(base + c * ch, ch)], sout[c % 2]
            )
            out_cp.start()
            out_cp.wait()
            if c + 1 < nch:
                start_in(c + 1)

    return body(memory, k_all)


def kernel(q, k, pos_set, neg_set, k_all, memory):
    neg_flat = neg_set.reshape(-1, _D)
    out = _logits(q, k, pos_set, neg_flat, memory)
    new_memory = _queue_update(memory, k_all)
    return (out, new_memory)


# speedup vs baseline: 1.0102x; 1.0102x over previous
import functools

import jax
import jax.numpy as jnp
from jax import lax
from jax.experimental import pallas as pl
from jax.experimental.pallas import tpu as pltpu
from jax.experimental.pallas import tpu_sc as plsc

_D = 128
_QUEUE = 65536
_INV_T = 10.0
_B = 256
_W = 8192
_HW = 2048
_NBLK = _QUEUE // _HW
_NCOL = 1 + 2048 + _QUEUE
_GRID = (_NCOL + _W - 1) // _W


def _out_body(q_ref, k_ref, pos_ref, neg_ref, memA_ref, memB_ref, memC_ref,
              memD_ref, out_ref, stash_ref):
    t = pl.program_id(0)
    q = q_ref[:]
    is0 = t == 0

    @pl.when(is0)
    def _():
        l_pos_k = jnp.sum(q * k_ref[:], axis=1, keepdims=True) * _INV_T
        l_pos_set = jnp.mean(
            jnp.sum(q[:, None, :] * pos_ref[:], axis=2), axis=1, keepdims=True
        ) * _INV_T
        stash_ref[:, 1:2] = l_pos_k
        stash_ref[:, 2:3] = l_pos_set

    srcA = jnp.where(is0, neg_ref[:], memA_ref[:])
    dn = (((1,), (1,)), ((), ()))
    pA = lax.dot_general(
        q, srcA, dn, preferred_element_type=jnp.float32) * _INV_T
    pB = lax.dot_general(
        q, memB_ref[:], dn, preferred_element_type=jnp.float32) * _INV_T
    pC = lax.dot_general(
        q, memC_ref[:], dn, preferred_element_type=jnp.float32) * _INV_T
    pD = lax.dot_general(
        q, memD_ref[:], dn, preferred_element_type=jnp.float32) * _INV_T
    p = jnp.concatenate([pA, pB, pC, pD], axis=1)
    rolled = pltpu.roll(p, shift=1, axis=1)

    out_ref[0:_B, :] = rolled
    out_ref[_B:2 * _B, :] = rolled
    prev = stash_ref[:, 0:1]
    out_ref[0:_B, 0:1] = jnp.where(is0, stash_ref[:, 1:2], prev)
    out_ref[_B:2 * _B, 0:1] = jnp.where(is0, stash_ref[:, 2:3], prev)
    stash_ref[:, 0:1] = rolled[:, 0:1]


def _logits(q, k, pos_set, neg_flat, memory):
    return pl.pallas_call(
        _out_body,
        grid=(_GRID,),
        in_specs=[
            pl.BlockSpec((_B, _D), lambda t: (0, 0)),
            pl.BlockSpec((_B, _D), lambda t: (0, 0)),
            pl.BlockSpec((_B, 4, _D), lambda t: (0, 0, 0)),
            pl.BlockSpec((_HW, _D), lambda t: (0, 0)),
            pl.BlockSpec(
                (_HW, _D), lambda t: (jnp.maximum(4 * t - 1, 3), 0)
            ),
            pl.BlockSpec(
                (_HW, _D), lambda t: (jnp.minimum(4 * t, _NBLK - 4), 0)
            ),
            pl.BlockSpec(
                (_HW, _D), lambda t: (jnp.minimum(4 * t + 1, _NBLK - 3), 0)
            ),
            pl.BlockSpec(
                (_HW, _D), lambda t: (jnp.minimum(4 * t + 2, _NBLK - 2), 0)
            ),
        ],
        out_specs=pl.BlockSpec((2 * _B, _W), lambda t: (0, t)),
        out_shape=jax.ShapeDtypeStruct((2 * _B, _NCOL), jnp.float32),
        scratch_shapes=[pltpu.VMEM((_B, _D), jnp.float32)],
    )(q, k, pos_set, neg_flat, memory, memory, memory, memory)


def _queue_update(memory, k_all):
    info = plsc.get_sparse_core_info()
    nw = info.num_cores * info.num_subcores
    rows_per = _QUEUE // nw
    ch = 256
    nch = rows_per // ch
    mesh = plsc.VectorSubcoreMesh(core_axis_name="c", subcore_axis_name="s")

    @functools.partial(
        pl.kernel,
        mesh=mesh,
        out_type=jax.ShapeDtypeStruct((_QUEUE, _D), jnp.float32),
        scratch_types=[
            pltpu.VMEM((ch, _D), jnp.float32),
            pltpu.VMEM((ch, _D), jnp.float32),
            pltpu.VMEM((ch, _D), jnp.float32),
            pltpu.SemaphoreType.DMA,
            pltpu.SemaphoreType.DMA,
            pltpu.SemaphoreType.DMA,
            pltpu.SemaphoreType.DMA,
            pltpu.SemaphoreType.DMA,
            pltpu.SemaphoreType.DMA,
        ],
    )
    def body(mem_hbm, kall_hbm, out_hbm, buf0, buf1, buf2,
             si0, si1, si2, so0, so1, so2):
        wid = lax.axis_index("s") * info.num_cores + lax.axis_index("c")
        base = wid * rows_per
        nbuf = 3
        bufs = (buf0, buf1, buf2)
        sin = (si0, si1, si2)
        sout = (so0, so1, so2)

        def start_in(c):
            b = bufs[c % nbuf]
            sem = sin[c % nbuf]
            if c * ch < _B:
                @pl.when(wid == 0)
                def _():
                    pltpu.make_async_copy(
                        kall_hbm.at[pl.ds(c * ch, ch)], b, sem
                    ).start()

                @pl.when(wid != 0)
                def _():
                    pltpu.make_async_copy(
                        mem_hbm.at[pl.ds(base + c * ch, ch)], b, sem
                    ).start()
            else:
                pltpu.make_async_copy(
                    mem_hbm.at[pl.ds(base + c * ch, ch)], b, sem
                ).start()

        def wait_out(c):
            pltpu.make_async_copy(
                bufs[c % nbuf], out_hbm.at[pl.ds(base + c * ch, ch)],
                sout[c % nbuf],
            ).wait()

        for c in range(nbuf):
            start_in(c)
        for c in range(nch):
            pltpu.make_async_copy(
                mem_hbm.at[pl.ds(base + c * ch, ch)], bufs[c % nbuf],
                sin[c % nbuf],
            ).wait()
            pltpu.make_async_copy(
                bufs[c % nbuf], out_hbm.at[pl.ds(base + c * ch, ch)],
                sout[c % nbuf],
            ).start()
            if 1 <= c and c + nbuf - 1 < nch:
                wait_out(c - 1)
                start_in(c + nbuf - 1)
        for c in range(max(nch - nbuf, 0), nch):
            wait_out(c)

    return body(memory, k_all)


def kernel(q, k, pos_set, neg_set, k_all, memory):
    neg_flat = neg_set.reshape(-1, _D)
    out = _logits(q, k, pos_set, neg_flat, memory)
    new_memory = _queue_update(memory, k_all)
    return (out, new_memory)
